# baseline (speedup 1.0000x reference)
import functools

import jax
import jax.numpy as jnp
from jax import lax
from jax.experimental import pallas as pl
from jax.experimental.pallas import tpu as pltpu
from jax.experimental.pallas import tpu_sc as plsc

B = 16384
S = 200
H = 64
NBT = B // 128
NS = 2
NW = 32
BT_PER_W = NBT // NW
NUNIT = S // NS


def _make_kernel():
    info = plsc.get_sparse_core_info()
    nc = info.num_cores
    mesh = plsc.VectorSubcoreMesh(core_axis_name="c", subcore_axis_name="s")

    @functools.partial(
        pl.kernel,
        mesh=mesh,
        out_type=jax.ShapeDtypeStruct((S, H // 8, NBT, 8, 128), jnp.float32),
        scratch_types=[
            pltpu.VMEM((128, S + 1), jnp.int32),
            pltpu.VMEM((2, NS, 128), jnp.int32),
            pltpu.VMEM((2, NS, 128, H), jnp.float32),
            pltpu.VMEM((2, NS, H // 8, 8, 129), jnp.float32),
            pltpu.SemaphoreType.DMA,
            pltpu.SemaphoreType.DMA,
        ],
        compiler_params=pltpu.CompilerParams(
            use_tc_tiling_on_sc=False, needs_layout_passes=False
        ),
    )
    def k(table_hbm, idx_hbm, out_hbm, idx_v, sidx, g_v, t_v, sem_g, sem_w):
        wid = lax.axis_index("s") * nc + lax.axis_index("c")
        iota = lax.iota(jnp.int32, 16)

        def stage_and_gather(buf, u):
            for sj in range(NS):
                scol = jnp.full((16,), u * NS + sj, jnp.int32)

                def stage_blk(blk, c, sj=sj):
                    col = plsc.load_gather(idx_v, [blk * 16 + iota, scol])
                    sidx[buf, sj, pl.ds(blk * 16, 16)] = col
                    return c

                lax.fori_loop(0, 8, stage_blk, 0, unroll=8)
            for sj in range(NS):
                pltpu.async_copy(
                    table_hbm.at[sidx.at[buf, sj]], g_v.at[buf, sj], sem_g)

        def wait_gathers(buf):
            for sj in range(NS):
                pltpu.make_async_copy(
                    table_hbm.at[sidx.at[buf, sj]], g_v.at[buf, sj], sem_g).wait()

        hvecs = []
        for h0 in range(0, H, 16):
            hv = h0 + iota
            hvecs.append((hv // 8, hv - (hv // 8) * 8))

        def transpose_unit(buf):
            for sj in range(NS):
                t3 = t_v.at[buf, sj]
                g2d = g_v.at[buf, sj]

                def do_bl(bl, c, t3=t3, g2d=g2d):
                    blv = jnp.full((16,), bl, jnp.int32)
                    for k, (htv, hlv) in enumerate(hvecs):
                        v = g2d[bl, pl.ds(k * 16, 16)]
                        plsc.store_scatter(t3, [htv, hlv, blv], v)
                    return c

                lax.fori_loop(0, 128, do_bl, 0, unroll=16)

        def drain_wb():
            pltpu.make_async_copy(
                t_v.at[0, :, :, :, pl.ds(0, 128)],
                out_hbm.at[pl.ds(0, NS), :, 0], sem_w).wait()

        def run_bt(t, carry):
            bt = wid * BT_PER_W + t
            pltpu.sync_copy(idx_hbm.at[pl.ds(bt * 128, 128)],
                            idx_v.at[:, pl.ds(0, S)])
            stage_and_gather(0, 0)

            def pair(p, carry2):
                for buf in range(2):
                    u = p * 2 + buf
                    wait_gathers(buf)

                    @pl.when(u + 1 < NUNIT)
                    def _(buf=buf, u=u):
                        stage_and_gather(1 - buf, u + 1)

                    @pl.when(u >= 2)
                    def _():
                        drain_wb()

                    transpose_unit(buf)
                    pltpu.async_copy(
                        t_v.at[buf, :, :, :, pl.ds(0, 128)],
                        out_hbm.at[pl.ds(u * NS, NS), :, bt], sem_w)
                return carry2

            lax.fori_loop(0, NUNIT // 2, pair, 0)
            drain_wb()
            drain_wb()
            return carry

        lax.fori_loop(0, BT_PER_W, run_bt, 0)

    return k


def kernel(position_ids, table):
    b, s = position_ids.shape
    idx = position_ids.astype(jnp.int32)
    five = _make_kernel()(table, idx)
    t = five.transpose(2, 4, 0, 1, 3)
    return t.reshape(b, s, H)


# speedup vs baseline: 1.0403x; 1.0403x over previous
import functools

import jax
import jax.numpy as jnp
from jax import lax
from jax.experimental import pallas as pl
from jax.experimental.pallas import tpu as pltpu
from jax.experimental.pallas import tpu_sc as plsc

B = 16384
S = 200
H = 64
NBT = B // 128
NS = 2
NW = 32
BT_PER_W = NBT // NW
NUNIT = S // NS


def _make_kernel():
    info = plsc.get_sparse_core_info()
    nc = info.num_cores
    mesh = plsc.VectorSubcoreMesh(core_axis_name="c", subcore_axis_name="s")

    @functools.partial(
        pl.kernel,
        mesh=mesh,
        out_type=jax.ShapeDtypeStruct((S, H // 8, NBT, 8, 128), jnp.float32),
        scratch_types=[
            pltpu.VMEM((128, S + 1), jnp.int32),
            pltpu.VMEM((2, NS, 128), jnp.int32),
            pltpu.VMEM((2, NS, 128, H), jnp.float32),
            pltpu.VMEM((2, NS, H // 8, 8, 129), jnp.float32),
            pltpu.SemaphoreType.DMA,
            pltpu.SemaphoreType.DMA,
        ],
        compiler_params=pltpu.CompilerParams(
            use_tc_tiling_on_sc=False, needs_layout_passes=False,
            disable_bounds_checks=True
        ),
    )
    def k(table_hbm, idx_hbm, out_hbm, idx_v, sidx, g_v, t_v, sem_g, sem_w):
        wid = lax.axis_index("s") * nc + lax.axis_index("c")
        iota = lax.iota(jnp.int32, 16)

        def stage_and_gather(buf, u):
            for sj in range(NS):
                scol = jnp.full((16,), u * NS + sj, jnp.int32)

                def stage_blk(blk, c, sj=sj):
                    col = plsc.load_gather(idx_v, [blk * 16 + iota, scol])
                    sidx[buf, sj, pl.ds(blk * 16, 16)] = col
                    return c

                lax.fori_loop(0, 8, stage_blk, 0, unroll=8)
            for sj in range(NS):
                pltpu.async_copy(
                    table_hbm.at[sidx.at[buf, sj]], g_v.at[buf, sj], sem_g)

        def wait_gathers(buf):
            for sj in range(NS):
                pltpu.make_async_copy(
                    table_hbm.at[sidx.at[buf, sj]], g_v.at[buf, sj], sem_g).wait()

        hvecs = []
        for h0 in range(0, H, 16):
            hv = h0 + iota
            hvecs.append((hv // 8, hv - (hv // 8) * 8))

        def transpose_unit(buf):
            for sj in range(NS):
                t3 = t_v.at[buf, sj]
                g2d = g_v.at[buf, sj]

                def do_bl(bl, c, t3=t3, g2d=g2d):
                    blv = jnp.full((16,), bl, jnp.int32)
                    for k, (htv, hlv) in enumerate(hvecs):
                        v = g2d[bl, pl.ds(k * 16, 16)]
                        plsc.store_scatter(t3, [htv, hlv, blv], v)
                    return c

                lax.fori_loop(0, 128, do_bl, 0, unroll=4)

        def drain_wb():
            pltpu.make_async_copy(
                t_v.at[0, :, :, :, pl.ds(0, 128)],
                out_hbm.at[pl.ds(0, NS), :, 0], sem_w).wait()

        def run_bt(t, carry):
            bt = wid * BT_PER_W + t
            pltpu.sync_copy(idx_hbm.at[pl.ds(bt * 128, 128)],
                            idx_v.at[:, pl.ds(0, S)])
            stage_and_gather(0, 0)

            def pair(p, carry2):
                for buf in range(2):
                    u = p * 2 + buf
                    wait_gathers(buf)

                    @pl.when(u + 1 < NUNIT)
                    def _(buf=buf, u=u):
                        stage_and_gather(1 - buf, u + 1)

                    @pl.when(u >= 2)
                    def _():
                        drain_wb()

                    transpose_unit(buf)
                    pltpu.async_copy(
                        t_v.at[buf, :, :, :, pl.ds(0, 128)],
                        out_hbm.at[pl.ds(u * NS, NS), :, bt], sem_w)
                return carry2

            lax.fori_loop(0, NUNIT // 2, pair, 0)
            drain_wb()
            drain_wb()
            return carry

        lax.fori_loop(0, BT_PER_W, run_bt, 0)

    return k


def kernel(position_ids, table):
    b, s = position_ids.shape
    idx = position_ids.astype(jnp.int32)
    five = _make_kernel()(table, idx)
    t = five.transpose(2, 4, 0, 1, 3)
    return t.reshape(b, s, H)


# speedup vs baseline: 1.8129x; 1.7426x over previous
import functools

import jax
import jax.numpy as jnp
from jax import lax
from jax.experimental import pallas as pl
from jax.experimental.pallas import tpu as pltpu
from jax.experimental.pallas import tpu_sc as plsc

B = 16384
S = 200
H = 64
NBT = B // 128
NS = 2
NW = 32
BT_PER_W = NBT // NW
NUNIT = S // NS


def _make_kernel():
    info = plsc.get_sparse_core_info()
    nc = info.num_cores
    mesh = plsc.VectorSubcoreMesh(core_axis_name="c", subcore_axis_name="s")

    @functools.partial(
        pl.kernel,
        mesh=mesh,
        out_type=jax.ShapeDtypeStruct((S, H // 8, NBT, 8, 128), jnp.float32),
        scratch_types=[
            pltpu.VMEM((128, S + 1), jnp.int32),
            pltpu.VMEM((2, NS, 128), jnp.int32),
            pltpu.VMEM((2, NS, 128, H), jnp.float32),
            pltpu.VMEM((2, NS, H // 8, 8, 129), jnp.float32),
            pltpu.SemaphoreType.DMA,
            pltpu.SemaphoreType.DMA,
        ],
        compiler_params=pltpu.CompilerParams(
            use_tc_tiling_on_sc=False, needs_layout_passes=False,
            disable_bounds_checks=True
        ),
    )
    def k(table_hbm, idx_hbm, out_hbm, idx_v, sidx, g_v, t_v, sem_g, sem_w):
        wid = lax.axis_index("s") * nc + lax.axis_index("c")
        iota = lax.iota(jnp.int32, 16)

        def stage_and_gather(buf, u):
            for sj in range(NS):
                scol = jnp.full((16,), u * NS + sj, jnp.int32)

                def stage_blk(blk, c, sj=sj):
                    col = plsc.load_gather(idx_v, [blk * 16 + iota, scol])
                    sidx[buf, sj, pl.ds(blk * 16, 16)] = col
                    return c

                lax.fori_loop(0, 8, stage_blk, 0, unroll=8)
            for sj in range(NS):
                pltpu.async_copy(
                    table_hbm.at[sidx.at[buf, sj]], g_v.at[buf, sj], sem_g)

        def wait_gathers(buf):
            for sj in range(NS):
                pltpu.make_async_copy(
                    table_hbm.at[sidx.at[buf, sj]], g_v.at[buf, sj], sem_g).wait()

        hvecs = []
        for h0 in range(0, H, 16):
            hv = h0 + iota
            hvecs.append((hv // 8, hv - (hv // 8) * 8))

        def transpose_unit(buf):
            for sj in range(NS):
                t3 = t_v.at[buf, sj]
                g2d = g_v.at[buf, sj]

                @plsc.parallel_loop(0, 128, unroll=4)
                def do_bl(bl, t3=t3, g2d=g2d):
                    blv = jnp.full((16,), bl, jnp.int32)
                    for k, (htv, hlv) in enumerate(hvecs):
                        v = g2d[bl, pl.ds(k * 16, 16)]
                        plsc.store_scatter(t3, [htv, hlv, blv], v)

        def drain_wb():
            pltpu.make_async_copy(
                t_v.at[0, :, :, :, pl.ds(0, 128)],
                out_hbm.at[pl.ds(0, NS), :, 0], sem_w).wait()

        def run_bt(t, carry):
            bt = wid * BT_PER_W + t
            pltpu.sync_copy(idx_hbm.at[pl.ds(bt * 128, 128)],
                            idx_v.at[:, pl.ds(0, S)])
            stage_and_gather(0, 0)

            def pair(p, carry2):
                for buf in range(2):
                    u = p * 2 + buf
                    wait_gathers(buf)

                    @pl.when(u + 1 < NUNIT)
                    def _(buf=buf, u=u):
                        stage_and_gather(1 - buf, u + 1)

                    @pl.when(u >= 2)
                    def _():
                        drain_wb()

                    transpose_unit(buf)
                    pltpu.async_copy(
                        t_v.at[buf, :, :, :, pl.ds(0, 128)],
                        out_hbm.at[pl.ds(u * NS, NS), :, bt], sem_w)
                return carry2

            lax.fori_loop(0, NUNIT // 2, pair, 0)
            drain_wb()
            drain_wb()
            return carry

        lax.fori_loop(0, BT_PER_W, run_bt, 0)

    return k


def kernel(position_ids, table):
    b, s = position_ids.shape
    idx = position_ids.astype(jnp.int32)
    five = _make_kernel()(table, idx)
    t = five.transpose(2, 4, 0, 1, 3)
    return t.reshape(b, s, H)
